# baseline (speedup 1.0000x reference)
import functools

import jax
import jax.numpy as jnp
from jax import lax
from jax.experimental import pallas as pl
from jax.experimental.pallas import tpu as pltpu
from jax.experimental.pallas import tpu_sc as plsc

BATCH = 16384
EMBED_K = 16
NUM_CORES = 2
NUM_SUBCORES = 16
NUM_WORKERS = NUM_CORES * NUM_SUBCORES
BPW = BATCH // NUM_WORKERS
NBUF = 16
CHUNKS = BPW // NBUF

_mesh = plsc.VectorSubcoreMesh(core_axis_name="c", subcore_axis_name="s")


@functools.partial(
    pl.kernel,
    mesh=_mesh,
    compiler_params=pltpu.CompilerParams(needs_layout_passes=False),
    out_type=jax.ShapeDtypeStruct((BATCH,), jnp.float32),
    scratch_types=[
        pltpu.VMEM((2 * BPW,), jnp.int32),
        pltpu.VMEM((BPW + 16,), jnp.int32),
        pltpu.VMEM((BPW + 16,), jnp.int32),
        pltpu.VMEM((16,), jnp.float32),
        pltpu.VMEM((NBUF, 2, 8, 128), jnp.float32),
        pltpu.VMEM((NBUF, 2, 8, 128), jnp.float32),
        pltpu.VMEM((16,), jnp.float32),
        pltpu.VMEM((BPW,), jnp.float32),
        pltpu.SemaphoreType.DMA,
        pltpu.SemaphoreType.DMA,
    ],
)
def _mf_sc_kernel(xf_hbm, wt_hbm, ht_hbm, out_hbm,
                  x_v, uidx_v, vidx_v, pad_lo, ublk, vblk, pad_hi, out_v,
                  sem_u, sem_v):
    wid = lax.axis_index("s") * NUM_CORES + lax.axis_index("c")
    base = wid * BPW

    pltpu.sync_copy(xf_hbm.at[pl.ds(2 * base, 2 * BPW)], x_v)
    iota = lax.iota(jnp.int32, 16)
    iota2 = iota * 2
    for g in range(BPW // 16):
        even = iota2 + (32 * g)
        uidx_v[pl.ds(g * 16, 16)] = plsc.load_gather(x_v, [even])
        vidx_v[pl.ds(g * 16, 16)] = plsc.load_gather(x_v, [even + 1])

    def _fire(j, b):
        u = uidx_v[pl.ds(j, 16)][0]
        v = vidx_v[pl.ds(j, 16)][0]
        u_off = pl.multiple_of((u >> 7) * 128, 128)
        v_off = pl.multiple_of((v >> 7) * 128, 128)
        for kt in range(2):
            pltpu.make_async_copy(
                wt_hbm.at[kt, :, pl.ds(u_off, 128)], ublk.at[b, kt], sem_u
            ).start()
            pltpu.make_async_copy(
                ht_hbm.at[kt, :, pl.ds(v_off, 128)], vblk.at[b, kt], sem_v
            ).start()

    for b in range(NBUF):
        _fire(b, b)

    def _chunk(c, _):
        j0 = c * NBUF
        outreg = jnp.zeros((16,), jnp.float32)
        for b in range(NBUF):
            j = j0 + b
            pltpu.make_async_copy(
                wt_hbm.at[:, :, pl.ds(0, 128)], ublk.at[b], sem_u
            ).wait()
            pltpu.make_async_copy(
                ht_hbm.at[:, :, pl.ds(0, 128)], vblk.at[b], sem_v
            ).wait()
            u = uidx_v[pl.ds(j, 16)][0]
            v = vidx_v[pl.ds(j, 16)][0]
            cu = (u & 127) - b
            cv = (v & 127) - b
            acc = jnp.zeros((16,), jnp.float32)
            for k in range(EMBED_K):
                acc = acc + (ublk[b, k // 8, k % 8, pl.ds(cu, 16)]
                             * vblk[b, k // 8, k % 8, pl.ds(cv, 16)])
            outreg = jnp.where(iota == b, acc, outreg)

            @pl.when(c < CHUNKS - 1)
            def _():
                _fire(j + NBUF, b)

        out_v[pl.ds(j0, 16)] = 1.0 / (1.0 + jnp.exp(-outreg))
        return ()

    lax.fori_loop(0, CHUNKS, _chunk, ())

    pltpu.sync_copy(out_v, out_hbm.at[pl.ds(base, BPW)])


def kernel(x, W, H):
    wt3 = W.T.reshape(2, 8, W.shape[0])
    ht3 = H.T.reshape(2, 8, H.shape[0])
    return _mf_sc_kernel(x.reshape(-1), wt3, ht3)
